# Initial kernel scaffold; baseline (speedup 1.0000x reference)
import jax
import jax.numpy as jnp
from jax.experimental import pallas as pl


def kernel(x, edge_feature, edge_index, W1, b1, W2, b2, Wq, bq, We, be, Wv, bv, A1, a1, A2, a2, P1, p1, P2, p2):
    raise NotImplementedError("write your pallas kernel here")


# speedup vs baseline: 1.0139x; 1.0139x over previous
import functools

import jax
import jax.numpy as jnp
import numpy as np
from jax.experimental import pallas as pl


_H = 8


def _edge_body(xi_ref, ef_ref, rev_ref, xj_ref,
               W1_ref, b1_ref, W2_ref, b2_ref,
               Wq_ref, bq_ref, We_ref, be_ref, Wv_ref, bv_ref,
               B1_ref, a1_ref, B2_ref, a2_ref, S_ref,
               gcn_ref, xatt_ref):
    xi = xi_ref[...]
    ef = ef_ref[...]
    rev = rev_ref[...]
    xj = xj_ref[...]
    f32 = jnp.float32

    cat = jnp.concatenate([xi, ef, rev, xj], axis=1)
    h = jnp.maximum(jnp.dot(cat, W1_ref[...], preferred_element_type=f32)
                    + b1_ref[...], 0.0)
    gcn_ref[...] = jnp.dot(h, W2_ref[...], preferred_element_type=f32) + b2_ref[...]

    q = jnp.dot(xi, Wq_ref[...], preferred_element_type=f32) + bq_ref[...]
    ep = jnp.dot(ef, We_ref[...], preferred_element_type=f32) + be_ref[...]
    q2 = jnp.concatenate([q, ep], axis=1)
    h1 = jnp.maximum(jnp.dot(q2, B1_ref[...], preferred_element_type=f32)
                     + a1_ref[...], 0.0)
    lg = jnp.dot(h1, B2_ref[...], preferred_element_type=f32) + a2_ref[...]
    m = jnp.max(lg, axis=-1, keepdims=True)
    ex = jnp.exp(lg - m)
    s8 = jnp.dot(ex, S_ref[...], preferred_element_type=f32)
    denom = jnp.concatenate([s8] * 16, axis=1)
    val = jnp.dot(xj, Wv_ref[...], preferred_element_type=f32) + bv_ref[...]
    xatt_ref[...] = (ex / denom) * val


def _node_body(x_ref, agg_ref, P1_ref, p1_ref, P2_ref, p2_ref, out_ref):
    f32 = jnp.float32
    cat = jnp.concatenate([x_ref[...], agg_ref[...]], axis=1)
    h = jnp.maximum(jnp.dot(cat, P1_ref[...], preferred_element_type=f32)
                    + p1_ref[...], 0.0)
    out_ref[...] = jnp.dot(h, P2_ref[...], preferred_element_type=f32) + p2_ref[...]


def _pick_tile(n, want):
    t = want
    while n % t or t % 8:
        t //= 2
        if t < 8:
            return 8
    return t


def kernel(x, edge_feature, edge_index, W1, b1, W2, b2, Wq, bq, We, be,
           Wv, bv, A1, a1, A2, a2, P1, p1, P2, p2):
    n = x.shape[0]
    e = edge_feature.shape[0]
    dn = x.shape[1]
    de = edge_feature.shape[1]
    da = Wv.shape[1]

    src = edge_index[0].astype(jnp.int32)
    dst = edge_index[1].astype(jnp.int32)

    keys = src * n + dst
    order = jnp.argsort(keys)
    skeys = keys[order]
    rkeys = dst * n + src
    pos = jnp.searchsorted(skeys, rkeys)
    posc = jnp.clip(pos, 0, e - 1)
    match = skeys[posc] == rkeys
    rev_idx = order[posc]

    xi = x[src]
    xj = x[dst]
    rev = jnp.where(match[:, None], edge_feature[rev_idx], 0.0)

    eye = jnp.eye(_H, dtype=jnp.float32)
    B1 = jnp.kron(A1.T, eye)
    a1r = jnp.repeat(a1, _H)
    B2 = jnp.kron(A2.T, eye)
    a2r = jnp.repeat(a2, _H)
    S = jnp.kron(jnp.ones((da // _H, 1), jnp.float32), eye)

    te = _pick_tile(e, 1600)
    grid_e = e // te

    def _full(shape):
        return pl.BlockSpec(shape, lambda i: (0,) * len(shape))

    row2 = lambda k: _full((1, k))

    edge_specs = [
        pl.BlockSpec((te, dn), lambda i: (i, 0)),
        pl.BlockSpec((te, de), lambda i: (i, 0)),
        pl.BlockSpec((te, de), lambda i: (i, 0)),
        pl.BlockSpec((te, dn), lambda i: (i, 0)),
        _full(W1.shape), row2(W1.shape[1]),
        _full(W2.shape), row2(W2.shape[1]),
        _full(Wq.shape), row2(Wq.shape[1]),
        _full(We.shape), row2(We.shape[1]),
        _full(Wv.shape), row2(Wv.shape[1]),
        _full(B1.shape), row2(B1.shape[1]),
        _full(B2.shape), row2(B2.shape[1]),
        _full(S.shape),
    ]
    gcn, xatt = pl.pallas_call(
        _edge_body,
        grid=(grid_e,),
        in_specs=edge_specs,
        out_specs=[pl.BlockSpec((te, de), lambda i: (i, 0)),
                   pl.BlockSpec((te, da), lambda i: (i, 0))],
        out_shape=[jax.ShapeDtypeStruct((e, de), jnp.float32),
                   jax.ShapeDtypeStruct((e, da), jnp.float32)],
    )(xi, edge_feature, rev, xj,
      W1, b1[None, :], W2, b2[None, :],
      Wq, bq[None, :], We, be[None, :], Wv, bv[None, :],
      B1, a1r[None, :], B2, a2r[None, :], S)

    agg = jax.ops.segment_max(xatt, src, num_segments=n)
    deg = jnp.zeros((n,), jnp.float32).at[src].add(1.0)
    agg = jnp.where(deg[:, None] > 0, agg, 0.0)

    tn = _pick_tile(n, 2000)
    out = pl.pallas_call(
        _node_body,
        grid=(n // tn,),
        in_specs=[
            pl.BlockSpec((tn, dn), lambda i: (i, 0)),
            pl.BlockSpec((tn, da), lambda i: (i, 0)),
            _full(P1.shape), row2(P1.shape[1]),
            _full(P2.shape), row2(P2.shape[1]),
        ],
        out_specs=pl.BlockSpec((tn, P2.shape[1]), lambda i: (i, 0)),
        out_shape=jax.ShapeDtypeStruct((n, P2.shape[1]), jnp.float32),
    )(x, agg, P1, p1[None, :], P2, p2[None, :])

    return (out, gcn)


# speedup vs baseline: 1.1497x; 1.1340x over previous
import functools

import jax
import jax.numpy as jnp
import numpy as np
from jax import lax
from jax.experimental import pallas as pl
from jax.experimental.pallas import tpu as pltpu
from jax.experimental.pallas import tpu_sc as plsc


_H = 8


def _sc_gather3(x, ef, src, dst, rev_idx):
    e, d = ef.shape[0], ef.shape[1]
    info = plsc.get_sparse_core_info()
    nc, ns = info.num_cores, info.num_subcores
    nw = nc * ns
    C = 128
    n_chunks = e // C
    mesh = plsc.VectorSubcoreMesh(core_axis_name="c", subcore_axis_name="s")

    @functools.partial(
        pl.kernel, mesh=mesh,
        out_type=[jax.ShapeDtypeStruct((e, d), jnp.float32)] * 3,
        scratch_types=[pltpu.VMEM((C,), jnp.int32),
                       pltpu.VMEM((C, d), jnp.float32),
                       pltpu.SemaphoreType.DMA],
    )
    def k(x_hbm, ef_hbm, src_hbm, dst_hbm, rid_hbm,
          xi_out, xj_out, rev_out, idx_v, rows_v, sem):
        wid = lax.axis_index("s") * nc + lax.axis_index("c")
        nj = (n_chunks - 1 - wid) // nw + 1

        def gather_one(table, idxlist, out):
            def body(j, carry):
                base = (wid + j * nw) * C
                pltpu.sync_copy(idxlist.at[pl.ds(base, C)], idx_v)
                pltpu.async_copy(table.at[idx_v], rows_v, sem).wait()
                pltpu.sync_copy(rows_v, out.at[pl.ds(base, C)])
                return carry
            lax.fori_loop(0, nj, body, 0)

        gather_one(x_hbm, src_hbm, xi_out)
        gather_one(x_hbm, dst_hbm, xj_out)
        gather_one(ef_hbm, rid_hbm, rev_out)

    return k(x, ef, src, dst, rev_idx)


def _edge_body(xi_ref, ef_ref, rev_ref, m_ref, xj_ref,
               W1_ref, b1_ref, W2_ref, b2_ref,
               Wq_ref, bq_ref, We_ref, be_ref, Wv_ref, bv_ref,
               B1_ref, a1_ref, B2_ref, a2_ref, S_ref,
               gcn_ref, xatt_ref):
    xi = xi_ref[...]
    ef = ef_ref[...]
    rev = rev_ref[...] * m_ref[...]
    xj = xj_ref[...]
    f32 = jnp.float32

    cat = jnp.concatenate([xi, ef, rev, xj], axis=1)
    h = jnp.maximum(jnp.dot(cat, W1_ref[...], preferred_element_type=f32)
                    + b1_ref[...], 0.0)
    gcn_ref[...] = jnp.dot(h, W2_ref[...], preferred_element_type=f32) + b2_ref[...]

    q = jnp.dot(xi, Wq_ref[...], preferred_element_type=f32) + bq_ref[...]
    ep = jnp.dot(ef, We_ref[...], preferred_element_type=f32) + be_ref[...]
    q2 = jnp.concatenate([q, ep], axis=1)
    h1 = jnp.maximum(jnp.dot(q2, B1_ref[...], preferred_element_type=f32)
                     + a1_ref[...], 0.0)
    lg = jnp.dot(h1, B2_ref[...], preferred_element_type=f32) + a2_ref[...]
    m = jnp.max(lg, axis=-1, keepdims=True)
    ex = jnp.exp(lg - m)
    s8 = jnp.dot(ex, S_ref[...], preferred_element_type=f32)
    denom = jnp.concatenate([s8] * 16, axis=1)
    val = jnp.dot(xj, Wv_ref[...], preferred_element_type=f32) + bv_ref[...]
    xatt_ref[...] = (ex / denom) * val


def _node_body(x_ref, agg_ref, P1_ref, p1_ref, P2_ref, p2_ref, out_ref):
    f32 = jnp.float32
    cat = jnp.concatenate([x_ref[...], agg_ref[...]], axis=1)
    h = jnp.maximum(jnp.dot(cat, P1_ref[...], preferred_element_type=f32)
                    + p1_ref[...], 0.0)
    out_ref[...] = jnp.dot(h, P2_ref[...], preferred_element_type=f32) + p2_ref[...]


def _pick_tile(n, want):
    t = want
    while n % t or t % 8:
        t //= 2
        if t < 8:
            return 8
    return t


def kernel(x, edge_feature, edge_index, W1, b1, W2, b2, Wq, bq, We, be,
           Wv, bv, A1, a1, A2, a2, P1, p1, P2, p2):
    n = x.shape[0]
    e = edge_feature.shape[0]
    dn = x.shape[1]
    de = edge_feature.shape[1]
    da = Wv.shape[1]

    src = edge_index[0].astype(jnp.int32)
    dst = edge_index[1].astype(jnp.int32)

    keys = src * n + dst
    order = jnp.argsort(keys)
    skeys = keys[order]
    rkeys = dst * n + src
    pos = jnp.searchsorted(skeys, rkeys)
    posc = jnp.clip(pos, 0, e - 1)
    match = skeys[posc] == rkeys
    rev_idx = order[posc]

    xi, xj, rev = _sc_gather3(x, edge_feature, src, dst, rev_idx)
    matchf = match.astype(jnp.float32)[:, None]

    eye = jnp.eye(_H, dtype=jnp.float32)
    B1 = jnp.kron(A1.T, eye)
    a1r = jnp.repeat(a1, _H)
    B2 = jnp.kron(A2.T, eye)
    a2r = jnp.repeat(a2, _H)
    S = jnp.kron(jnp.ones((da // _H, 1), jnp.float32), eye)

    te = _pick_tile(e, 1600)
    grid_e = e // te

    def _full(shape):
        return pl.BlockSpec(shape, lambda i: (0,) * len(shape))

    row2 = lambda k: _full((1, k))

    edge_specs = [
        pl.BlockSpec((te, dn), lambda i: (i, 0)),
        pl.BlockSpec((te, de), lambda i: (i, 0)),
        pl.BlockSpec((te, de), lambda i: (i, 0)),
        pl.BlockSpec((te, 1), lambda i: (i, 0)),
        pl.BlockSpec((te, dn), lambda i: (i, 0)),
        _full(W1.shape), row2(W1.shape[1]),
        _full(W2.shape), row2(W2.shape[1]),
        _full(Wq.shape), row2(Wq.shape[1]),
        _full(We.shape), row2(We.shape[1]),
        _full(Wv.shape), row2(Wv.shape[1]),
        _full(B1.shape), row2(B1.shape[1]),
        _full(B2.shape), row2(B2.shape[1]),
        _full(S.shape),
    ]
    gcn, xatt = pl.pallas_call(
        _edge_body,
        grid=(grid_e,),
        in_specs=edge_specs,
        out_specs=[pl.BlockSpec((te, de), lambda i: (i, 0)),
                   pl.BlockSpec((te, da), lambda i: (i, 0))],
        out_shape=[jax.ShapeDtypeStruct((e, de), jnp.float32),
                   jax.ShapeDtypeStruct((e, da), jnp.float32)],
    )(xi, edge_feature, rev, matchf, xj,
      W1, b1[None, :], W2, b2[None, :],
      Wq, bq[None, :], We, be[None, :], Wv, bv[None, :],
      B1, a1r[None, :], B2, a2r[None, :], S)

    agg = jax.ops.segment_max(xatt, src, num_segments=n)
    deg = jnp.zeros((n,), jnp.float32).at[src].add(1.0)
    agg = jnp.where(deg[:, None] > 0, agg, 0.0)

    tn = _pick_tile(n, 2000)
    out = pl.pallas_call(
        _node_body,
        grid=(n // tn,),
        in_specs=[
            pl.BlockSpec((tn, dn), lambda i: (i, 0)),
            pl.BlockSpec((tn, da), lambda i: (i, 0)),
            _full(P1.shape), row2(P1.shape[1]),
            _full(P2.shape), row2(P2.shape[1]),
        ],
        out_specs=pl.BlockSpec((tn, P2.shape[1]), lambda i: (i, 0)),
        out_shape=jax.ShapeDtypeStruct((n, P2.shape[1]), jnp.float32),
    )(x, agg, P1, p1[None, :], P2, p2[None, :])

    return (out, gcn)


# speedup vs baseline: 1.1686x; 1.0165x over previous
import functools

import jax
import jax.numpy as jnp
import numpy as np
from jax import lax
from jax.experimental import pallas as pl
from jax.experimental.pallas import tpu as pltpu
from jax.experimental.pallas import tpu_sc as plsc


_H = 8


def _sc_gather3(x, ef, src, dst, rev_idx):
    e, d = ef.shape[0], ef.shape[1]
    info = plsc.get_sparse_core_info()
    nc, ns = info.num_cores, info.num_subcores
    nw = nc * ns
    C = 128
    n_chunks = e // C
    mesh = plsc.VectorSubcoreMesh(core_axis_name="c", subcore_axis_name="s")

    @functools.partial(
        pl.kernel, mesh=mesh,
        out_type=[jax.ShapeDtypeStruct((e, d), jnp.float32)] * 3,
        scratch_types=[pltpu.VMEM((C,), jnp.int32),
                       pltpu.VMEM((C, d), jnp.float32),
                       pltpu.SemaphoreType.DMA],
    )
    def k(x_hbm, ef_hbm, src_hbm, dst_hbm, rid_hbm,
          xi_out, xj_out, rev_out, idx_v, rows_v, sem):
        wid = lax.axis_index("s") * nc + lax.axis_index("c")
        nj = (n_chunks - 1 - wid) // nw + 1

        def gather_one(table, idxlist, out):
            def body(j, carry):
                base = (wid + j * nw) * C
                pltpu.sync_copy(idxlist.at[pl.ds(base, C)], idx_v)
                pltpu.async_copy(table.at[idx_v], rows_v, sem).wait()
                pltpu.sync_copy(rows_v, out.at[pl.ds(base, C)])
                return carry
            lax.fori_loop(0, nj, body, 0)

        gather_one(x_hbm, src_hbm, xi_out)
        gather_one(x_hbm, dst_hbm, xj_out)
        gather_one(ef_hbm, rid_hbm, rev_out)

    return k(x, ef, src, dst, rev_idx)


def _edge_body(xi_ref, ef_ref, rev_ref, m_ref, xj_ref,
               W1_ref, b1_ref, W2_ref, b2_ref,
               Wq_ref, bq_ref, We_ref, be_ref, Wv_ref, bv_ref,
               B1_ref, a1_ref, B2_ref, a2_ref, S_ref,
               gcn_ref, xatt_ref):
    xi = xi_ref[...]
    ef = ef_ref[...]
    rev = rev_ref[...] * m_ref[...]
    xj = xj_ref[...]
    f32 = jnp.float32

    cat = jnp.concatenate([xi, ef, rev, xj], axis=1)
    h = jnp.maximum(jnp.dot(cat, W1_ref[...], preferred_element_type=f32)
                    + b1_ref[...], 0.0)
    gcn_ref[...] = jnp.dot(h, W2_ref[...], preferred_element_type=f32) + b2_ref[...]

    q = jnp.dot(xi, Wq_ref[...], preferred_element_type=f32) + bq_ref[...]
    ep = jnp.dot(ef, We_ref[...], preferred_element_type=f32) + be_ref[...]
    q2 = jnp.concatenate([q, ep], axis=1)
    h1 = jnp.maximum(jnp.dot(q2, B1_ref[...], preferred_element_type=f32)
                     + a1_ref[...], 0.0)
    lg = jnp.dot(h1, B2_ref[...], preferred_element_type=f32) + a2_ref[...]
    m = jnp.max(lg, axis=-1, keepdims=True)
    ex = jnp.exp(lg - m)
    s8 = jnp.dot(ex, S_ref[...], preferred_element_type=f32)
    denom = jnp.concatenate([s8] * 16, axis=1)
    val = jnp.dot(xj, Wv_ref[...], preferred_element_type=f32) + bv_ref[...]
    xatt_ref[...] = (ex / denom) * val


def _node_body(x_ref, agg_ref, P1_ref, p1_ref, P2_ref, p2_ref, out_ref):
    f32 = jnp.float32
    agg = agg_ref[...]
    agg = jnp.where(agg == -jnp.inf, 0.0, agg)
    cat = jnp.concatenate([x_ref[...], agg], axis=1)
    h = jnp.maximum(jnp.dot(cat, P1_ref[...], preferred_element_type=f32)
                    + p1_ref[...], 0.0)
    out_ref[...] = jnp.dot(h, P2_ref[...], preferred_element_type=f32) + p2_ref[...]


def _pick_tile(n, want):
    t = want
    while n % t or t % 8:
        t //= 2
        if t < 8:
            return 8
    return t


def kernel(x, edge_feature, edge_index, W1, b1, W2, b2, Wq, bq, We, be,
           Wv, bv, A1, a1, A2, a2, P1, p1, P2, p2):
    n = x.shape[0]
    e = edge_feature.shape[0]
    dn = x.shape[1]
    de = edge_feature.shape[1]
    da = Wv.shape[1]

    src = edge_index[0].astype(jnp.int32)
    dst = edge_index[1].astype(jnp.int32)

    keys = src * n + dst
    iota = jnp.arange(e, dtype=jnp.int32)
    skeys, order = lax.sort_key_val(keys, iota)
    rkeys = dst * n + src
    pos = jnp.searchsorted(skeys, rkeys)
    posc = jnp.clip(pos, 0, e - 1)
    packed = jnp.stack([skeys, order], axis=1)
    g = packed[posc]
    match = g[:, 0] == rkeys
    rev_idx = g[:, 1]

    xi, xj, rev = _sc_gather3(x, edge_feature, src, dst, rev_idx)
    matchf = match.astype(jnp.float32)[:, None]

    eye = jnp.eye(_H, dtype=jnp.float32)
    B1 = jnp.kron(A1.T, eye)
    a1r = jnp.repeat(a1, _H)
    B2 = jnp.kron(A2.T, eye)
    a2r = jnp.repeat(a2, _H)
    S = jnp.kron(jnp.ones((da // _H, 1), jnp.float32), eye)

    te = _pick_tile(e, 1600)
    grid_e = e // te

    def _full(shape):
        return pl.BlockSpec(shape, lambda i: (0,) * len(shape))

    row2 = lambda k: _full((1, k))

    edge_specs = [
        pl.BlockSpec((te, dn), lambda i: (i, 0)),
        pl.BlockSpec((te, de), lambda i: (i, 0)),
        pl.BlockSpec((te, de), lambda i: (i, 0)),
        pl.BlockSpec((te, 1), lambda i: (i, 0)),
        pl.BlockSpec((te, dn), lambda i: (i, 0)),
        _full(W1.shape), row2(W1.shape[1]),
        _full(W2.shape), row2(W2.shape[1]),
        _full(Wq.shape), row2(Wq.shape[1]),
        _full(We.shape), row2(We.shape[1]),
        _full(Wv.shape), row2(Wv.shape[1]),
        _full(B1.shape), row2(B1.shape[1]),
        _full(B2.shape), row2(B2.shape[1]),
        _full(S.shape),
    ]
    gcn, xatt = pl.pallas_call(
        _edge_body,
        grid=(grid_e,),
        in_specs=edge_specs,
        out_specs=[pl.BlockSpec((te, de), lambda i: (i, 0)),
                   pl.BlockSpec((te, da), lambda i: (i, 0))],
        out_shape=[jax.ShapeDtypeStruct((e, de), jnp.float32),
                   jax.ShapeDtypeStruct((e, da), jnp.float32)],
    )(xi, edge_feature, rev, matchf, xj,
      W1, b1[None, :], W2, b2[None, :],
      Wq, bq[None, :], We, be[None, :], Wv, bv[None, :],
      B1, a1r[None, :], B2, a2r[None, :], S)

    agg = jax.ops.segment_max(xatt, src, num_segments=n)

    tn = _pick_tile(n, 2000)
    out = pl.pallas_call(
        _node_body,
        grid=(n // tn,),
        in_specs=[
            pl.BlockSpec((tn, dn), lambda i: (i, 0)),
            pl.BlockSpec((tn, da), lambda i: (i, 0)),
            _full(P1.shape), row2(P1.shape[1]),
            _full(P2.shape), row2(P2.shape[1]),
        ],
        out_specs=pl.BlockSpec((tn, P2.shape[1]), lambda i: (i, 0)),
        out_shape=jax.ShapeDtypeStruct((n, P2.shape[1]), jnp.float32),
    )(x, agg, P1, p1[None, :], P2, p2[None, :])

    return (out, gcn)
